# Initial kernel scaffold; baseline (speedup 1.0000x reference)
import jax
import jax.numpy as jnp
from jax.experimental import pallas as pl


def kernel(x, edge_index, edge_attr, emb, Wr1, Wroot1, b1, Wr2, Wroot2, b2, Wr3, Wroot3, b3):
    raise NotImplementedError("write your pallas kernel here")


# speedup vs baseline: 7.8625x; 7.8625x over previous
import numpy as np

import jax
import jax.numpy as jnp
from jax import lax
from jax.experimental import pallas as pl
from jax.experimental.pallas import tpu as pltpu
from jax.experimental.pallas import tpu_sc as plsc

N = 10000
E = 320000
VOCAB = 64
D = 128
R = 4

NC = 2
NS = 16
L = 16
NW = NC * NS

K = 80
EW = E // NW
EA = E // NS
ROWS_W = 624
ROWS_TAIL = N - NS * ROWS_W
HROWS = 384
HR_W = HROWS // NS

def _lane_iota():
    return lax.iota(jnp.int32, L)


def _mesh():
    return plsc.VectorSubcoreMesh(
        core_axis_name="c", subcore_axis_name="s", num_cores=NC, num_subcores=NS
    )


_SC_PARAMS = pltpu.CompilerParams(needs_layout_passes=False)


_TAKE_DN = lax.GatherDimensionNumbers(offset_dims=(), collapsed_slice_dims=(0,),
                                      start_index_map=(0,))


def _take16(vec16, j):
    idx = jnp.full((L, 1), j, jnp.int32)
    return lax.gather(vec16, idx, _TAKE_DN, slice_sizes=(1,),
                      mode=lax.GatherScatterMode.PROMISE_IN_BOUNDS)


def _scale_body(cidx_hbm, s_hbm, hist_sp, stage_v, cidx_v, rowidx_v, sv_v, inv_v, sem):
    del sem
    c = lax.axis_index("c")
    s = lax.axis_index("s")

    @pl.when(c == 0)
    def _():
        zero16 = jnp.zeros((L,), jnp.float32)
        for j in range(HR_W):
            for i8 in range(D // L):
                stage_v[j, pl.ds(i8 * L, L)] = zero16
        pltpu.sync_copy(stage_v.at[pl.ds(0, HR_W)], hist_sp.at[pl.ds(s * HR_W, HR_W)])

    plsc.subcore_barrier()

    @pl.when(c == 0)
    def _():
        def hist_step(i, _):
            base = s * EA + i * K
            pltpu.sync_copy(cidx_hbm.at[pl.ds(base, K)], cidx_v)
            iota16 = _lane_iota()
            for jj in range(K // L):
                c16 = cidx_v[pl.ds(jj * L, L)]
                rowidx_v[pl.ds(jj * L, L)] = c16 >> 7
                for t in range(L):
                    cj = _take16(c16, t)
                    vhi = (cj >> 4) & 7
                    oh = jnp.where(iota16 == (cj & 15), 1.0, 0.0)
                    for i8 in range(D // L):
                        stage_v[jj * L + t, pl.ds(i8 * L, L)] = (
                            jnp.where(vhi == i8, oh, 0.0))
            pltpu.sync_copy(stage_v, hist_sp.at[rowidx_v], add=True)
            return _

        lax.fori_loop(0, EA // K, hist_step, None)

    plsc.subcore_barrier()

    @pl.when(c == 0)
    def _():
        off = s * HR_W
        pltpu.sync_copy(hist_sp.at[pl.ds(off, HR_W)], stage_v.at[pl.ds(0, HR_W)])
        for j in range(HR_W):
            for i8 in range(D // L):
                sl = pl.ds(i8 * L, L)
                stage_v[j, sl] = 1.0 / jnp.maximum(stage_v[j, sl], 1.0)
        pltpu.sync_copy(stage_v.at[pl.ds(0, HR_W)], hist_sp.at[pl.ds(off, HR_W)])

    plsc.subcore_barrier()

    @pl.when(c == 0)
    def _():
        pltpu.sync_copy(hist_sp, inv_v)

        def gather_step(i, _):
            base = s * EA + i * K
            pltpu.sync_copy(cidx_hbm.at[pl.ds(base, K)], cidx_v)
            for jj in range(K // L):
                c16 = cidx_v[pl.ds(jj * L, L)]
                sv_v[pl.ds(jj * L, L)] = plsc.load_gather(inv_v, [c16 >> 7, c16 & 127])
            pltpu.sync_copy(sv_v, s_hbm.at[pl.ds(base, K)])
            return _

        lax.fori_loop(0, EA // K, gather_step, None)


_scale_pass = pl.kernel(
    _scale_body,
    out_type=jax.ShapeDtypeStruct((E,), jnp.float32),
    mesh=_mesh(),
    scratch_types=[
        pltpu.VMEM_SHARED((HROWS, D), jnp.float32),
        pltpu.VMEM((K, D), jnp.float32),
        pltpu.VMEM((K,), jnp.int32),
        pltpu.VMEM((K,), jnp.int32),
        pltpu.VMEM((K,), jnp.float32),
        pltpu.VMEM((HROWS, D), jnp.float32),
        pltpu.SemaphoreType.DMA,
    ],
    compiler_params=_SC_PARAMS,
)


def _edge_body(hr_hbm, gidx_hbm, didx_hbm, s_hbm, zeros_hbm, accp_hbm,
               acc_sp, rows_v, gidx_v, didx_v, s_v, sem):
    c = lax.axis_index("c")
    s = lax.axis_index("s")
    wid = c * NS + s

    pltpu.sync_copy(zeros_hbm.at[pl.ds(0, ROWS_W)], acc_sp.at[pl.ds(s * ROWS_W, ROWS_W)])

    @pl.when(s == NS - 1)
    def _():
        pltpu.sync_copy(zeros_hbm.at[pl.ds(0, ROWS_TAIL)],
                        acc_sp.at[pl.ds(NS * ROWS_W, ROWS_TAIL)])

    plsc.subcore_barrier()

    def step(i, _):
        base = wid * EW + i * K
        pltpu.sync_copy(gidx_hbm.at[pl.ds(base, K)], gidx_v)
        pltpu.sync_copy(didx_hbm.at[pl.ds(base, K)], didx_v)
        pltpu.sync_copy(s_hbm.at[pl.ds(base, K)], s_v)
        pltpu.async_copy(hr_hbm.at[gidx_v], rows_v, sem).wait()
        for jj in range(K // L):
            s16 = s_v[pl.ds(jj * L, L)]
            for t in range(L):
                sj = _take16(s16, t)
                for i8 in range(D // L):
                    sl = pl.ds(i8 * L, L)
                    rows_v[jj * L + t, sl] = rows_v[jj * L + t, sl] * sj
        pltpu.sync_copy(rows_v, acc_sp.at[didx_v], add=True)
        return _

    lax.fori_loop(0, EW // K, step, None)
    plsc.subcore_barrier()
    pltpu.sync_copy(acc_sp.at[pl.ds(s * ROWS_W, ROWS_W)],
                    accp_hbm.at[c, pl.ds(s * ROWS_W, ROWS_W)])

    @pl.when(s == NS - 1)
    def _():
        pltpu.sync_copy(acc_sp.at[pl.ds(NS * ROWS_W, ROWS_TAIL)],
                        accp_hbm.at[c, pl.ds(NS * ROWS_W, ROWS_TAIL)])


_edge_pass = pl.kernel(
    _edge_body,
    out_type=jax.ShapeDtypeStruct((NC, N, D), jnp.float32),
    mesh=_mesh(),
    scratch_types=[
        pltpu.VMEM_SHARED((N, D), jnp.float32),
        pltpu.VMEM((K, D), jnp.float32),
        pltpu.VMEM((K,), jnp.int32),
        pltpu.VMEM((K,), jnp.int32),
        pltpu.VMEM((K,), jnp.float32),
        pltpu.SemaphoreType.DMA,
    ],
    compiler_params=_SC_PARAMS,
)


BN = 1000


def _mm(a, b):
    return jnp.dot(a, b, preferred_element_type=jnp.float32)


def _transform(h, wroot_ref, b_ref, wr_ref, h_ref, out0_ref, hr_ref):
    h_ref[...] = h
    out0_ref[...] = _mm(h, wroot_ref[...]) + b_ref[...]
    for r in range(R):
        hr_ref[r] = _mm(h, wr_ref[r])


def _embed_body(x_ref, emb_ref, wroot_ref, b_ref, wr_ref, h_ref, out0_ref, hr_ref):
    xb = x_ref[...]
    oh = jnp.where(xb == lax.broadcasted_iota(jnp.int32, (BN, VOCAB), 1), 1.0, 0.0)
    h = _mm(oh, emb_ref[...])
    _transform(h, wroot_ref, b_ref, wr_ref, h_ref, out0_ref, hr_ref)


def _layer_body(hprev_ref, out0prev_ref, acc_ref, wroot_ref, b_ref, wr_ref,
                h_ref, out0_ref, hr_ref):
    conv = out0prev_ref[...] + acc_ref[0] + acc_ref[1]
    h = jnp.maximum(conv, 0.0) + hprev_ref[...]
    _transform(h, wroot_ref, b_ref, wr_ref, h_ref, out0_ref, hr_ref)


def _final_body(hprev_ref, out0_ref, acc_ref, o_ref):
    conv = out0_ref[...] + acc_ref[0] + acc_ref[1]
    o_ref[...] = jnp.maximum(conv, 0.0) + hprev_ref[...]


_full2 = pl.BlockSpec((VOCAB, D), lambda i: (0, 0))
_wroot_spec = pl.BlockSpec((D, D), lambda i: (0, 0))
_b_spec = pl.BlockSpec((1, D), lambda i: (0, 0))
_wr_spec = pl.BlockSpec((R, D, D), lambda i: (0, 0, 0))
_nd_spec = pl.BlockSpec((BN, D), lambda i: (i, 0))
_hr_spec = pl.BlockSpec((R, BN, D), lambda i: (0, i, 0))
_acc_spec = pl.BlockSpec((NC, BN, D), lambda i: (0, i, 0))
_x_spec = pl.BlockSpec((BN, 1), lambda i: (i, 0))

_nd_t = jax.ShapeDtypeStruct((N, D), jnp.float32)
_hr_t = jax.ShapeDtypeStruct((R, N, D), jnp.float32)


def _embed_transform(x32, emb, wroot, b2, wr):
    return pl.pallas_call(
        _embed_body,
        grid=(N // BN,),
        in_specs=[_x_spec, _full2, _wroot_spec, _b_spec, _wr_spec],
        out_specs=[_nd_spec, _nd_spec, _hr_spec],
        out_shape=[_nd_t, _nd_t, _hr_t],
    )(x32, emb, wroot, b2, wr)


def _layer_transform(hprev, out0prev, accp, wroot, b2, wr):
    return pl.pallas_call(
        _layer_body,
        grid=(N // BN,),
        in_specs=[_nd_spec, _nd_spec, _acc_spec, _wroot_spec, _b_spec, _wr_spec],
        out_specs=[_nd_spec, _nd_spec, _hr_spec],
        out_shape=[_nd_t, _nd_t, _hr_t],
    )(hprev, out0prev, accp, wroot, b2, wr)


def _final(hprev, out0, accp):
    return pl.pallas_call(
        _final_body,
        grid=(N // BN,),
        in_specs=[_nd_spec, _nd_spec, _acc_spec],
        out_specs=_nd_spec,
        out_shape=_nd_t,
    )(hprev, out0, accp)


def kernel(x, edge_index, edge_attr, emb, Wr1, Wroot1, b1, Wr2, Wroot2, b2,
           Wr3, Wroot3, b3):
    x32 = x.astype(jnp.int32)
    src = edge_index[0].astype(jnp.int32)
    dst = edge_index[1].astype(jnp.int32)
    rel = edge_attr.astype(jnp.int32)
    gidx = rel * N + src
    cidx = dst * R + rel
    zeros_m = jnp.zeros((ROWS_W, D), jnp.float32)

    s_e = _scale_pass(cidx)

    h1, out0_1, hr1 = _embed_transform(x32, emb, Wroot1, b1.reshape(1, D), Wr1)
    acc1 = _edge_pass(hr1.reshape(R * N, D), gidx, dst, s_e, zeros_m)
    h2, out0_2, hr2 = _layer_transform(h1, out0_1, acc1, Wroot2, b2.reshape(1, D), Wr2)
    acc2 = _edge_pass(hr2.reshape(R * N, D), gidx, dst, s_e, zeros_m)
    h3, out0_3, hr3 = _layer_transform(h2, out0_2, acc2, Wroot3, b3.reshape(1, D), Wr3)
    acc3 = _edge_pass(hr3.reshape(R * N, D), gidx, dst, s_e, zeros_m)
    return _final(h3, out0_3, acc3)


# speedup vs baseline: 15.3654x; 1.9543x over previous
import numpy as np

import jax
import jax.numpy as jnp
from jax import lax
from jax.experimental import pallas as pl
from jax.experimental.pallas import tpu as pltpu
from jax.experimental.pallas import tpu_sc as plsc

N = 10000
E = 320000
VOCAB = 64
D = 128
R = 4

NC = 2
NS = 16
L = 16
NW = NC * NS

K = 80
EW = E // NW
CH = EW // K
EA = E // NS
ROWS_W = 624
ROWS_TAIL = N - NS * ROWS_W
HROWS = 320
HR_W = 40

def _lane_iota():
    return lax.iota(jnp.int32, L)


def _mesh():
    return plsc.VectorSubcoreMesh(
        core_axis_name="c", subcore_axis_name="s", num_cores=NC, num_subcores=NS
    )


_SC_PARAMS = pltpu.CompilerParams(needs_layout_passes=False)


_TAKE_DN = lax.GatherDimensionNumbers(offset_dims=(), collapsed_slice_dims=(0,),
                                      start_index_map=(0,))


def _take16(vec16, j):
    idx = jnp.full((L, 1), j, jnp.int32)
    return lax.gather(vec16, idx, _TAKE_DN, slice_sizes=(1,),
                      mode=lax.GatherScatterMode.PROMISE_IN_BOUNDS)


def _scale_body(cidx_hbm, zeros_hbm, s_hbm, hist_sp, hist_v, stage_v, cidx_v,
                rowid_v, sv_v, sem):
    del sem
    c = lax.axis_index("c")
    s = lax.axis_index("s")

    @pl.when(c == 0)
    def _():
        pltpu.sync_copy(zeros_hbm.at[pl.ds(0, HROWS)], hist_v)

        @pl.when(s < NS // 2)
        def _():
            pltpu.sync_copy(zeros_hbm.at[pl.ds(0, HR_W)],
                            hist_sp.at[pl.ds(s * HR_W, HR_W)])

        ones = jnp.ones((L,), jnp.float32)

        def hist_step(i, _):
            pltpu.sync_copy(cidx_hbm.at[pl.ds(s * EA + i * K, K)], cidx_v)
            for jj in range(K // L):
                c16 = cidx_v[pl.ds(jj * L, L)]
                plsc.addupdate_scatter(hist_v, [c16 >> 7, c16 & 127], ones)
            return _

        lax.fori_loop(0, EA // K, hist_step, None)

    plsc.subcore_barrier()

    @pl.when(c == 0)
    def _():
        for kk in range(HROWS // K):
            base = kk * K
            for jj in range(K // L):
                rowid_v[pl.ds(jj * L, L)] = _lane_iota() + (base + jj * L)
            pltpu.sync_copy(hist_v.at[pl.ds(base, K)], hist_sp.at[rowid_v],
                            add=True)

    plsc.subcore_barrier()

    @pl.when((c == 0) & (s < NS // 2))
    def _():
        off = s * HR_W
        pltpu.sync_copy(hist_sp.at[pl.ds(off, HR_W)], stage_v)
        for j in range(HR_W):
            for i8 in range(D // L):
                sl = pl.ds(i8 * L, L)
                stage_v[j, sl] = 1.0 / jnp.maximum(stage_v[j, sl], 1.0)
        pltpu.sync_copy(stage_v, hist_sp.at[pl.ds(off, HR_W)])

    plsc.subcore_barrier()

    @pl.when(c == 0)
    def _():
        pltpu.sync_copy(hist_sp, hist_v)

        def gather_step(i, _):
            base = s * EA + i * K
            pltpu.sync_copy(cidx_hbm.at[pl.ds(base, K)], cidx_v)
            for jj in range(K // L):
                c16 = cidx_v[pl.ds(jj * L, L)]
                sv_v[pl.ds(jj * L, L)] = plsc.load_gather(hist_v,
                                                          [c16 >> 7, c16 & 127])
            pltpu.sync_copy(sv_v, s_hbm.at[pl.ds(base, K)])
            return _

        lax.fori_loop(0, EA // K, gather_step, None)


_scale_pass = pl.kernel(
    _scale_body,
    out_type=jax.ShapeDtypeStruct((E,), jnp.float32),
    mesh=_mesh(),
    scratch_types=[
        pltpu.VMEM_SHARED((HROWS, D), jnp.float32),
        pltpu.VMEM((HROWS, D), jnp.float32),
        pltpu.VMEM((HR_W, D), jnp.float32),
        pltpu.VMEM((K,), jnp.int32),
        pltpu.VMEM((K,), jnp.int32),
        pltpu.VMEM((K,), jnp.float32),
        pltpu.SemaphoreType.DMA,
    ],
    compiler_params=_SC_PARAMS,
)


def _edge_body(hr_hbm, gidx_hbm, didx_hbm, s_hbm, zeros_hbm, accp_hbm,
               acc_sp, rows_a, rows_b, gidx_v, didx_c, s_v,
               sem_a, sem_b):
    c = lax.axis_index("c")
    s = lax.axis_index("s")
    wid = c * NS + s

    pltpu.sync_copy(zeros_hbm.at[pl.ds(0, ROWS_W)], acc_sp.at[pl.ds(s * ROWS_W, ROWS_W)])

    @pl.when(s == NS - 1)
    def _():
        pltpu.sync_copy(zeros_hbm.at[pl.ds(0, ROWS_TAIL)],
                        acc_sp.at[pl.ds(NS * ROWS_W, ROWS_TAIL)])

    pltpu.sync_copy(gidx_hbm.at[pl.ds(wid * EW, EW)], gidx_v)
    pltpu.sync_copy(s_hbm.at[pl.ds(wid * EW, EW)], s_v)
    plsc.subcore_barrier()

    pltpu.async_copy(hr_hbm.at[gidx_v.at[pl.ds(0, K)]], rows_a, sem_a)

    def process(cur, nxt, sem_cur, sem_nxt, i):
        pltpu.sync_copy(didx_hbm.at[pl.ds(wid * EW + i * K, K)], didx_c)
        pltpu.make_async_copy(zeros_hbm.at[pl.ds(0, K)], cur, sem_cur).wait()

        @pl.when(i + 1 < CH)
        def _():
            pltpu.async_copy(hr_hbm.at[gidx_v.at[pl.ds((i + 1) * K, K)]],
                             nxt, sem_nxt)

        for jj in range(K // L):
            s16 = s_v[pl.ds(i * K + jj * L, L)]
            for t in range(L):
                sj = _take16(s16, t)
                for i8 in range(D // L):
                    sl = pl.ds(i8 * L, L)
                    cur[jj * L + t, sl] = cur[jj * L + t, sl] * sj
        pltpu.sync_copy(cur, acc_sp.at[didx_c], add=True)

    def step(i, _):
        @pl.when(i % 2 == 0)
        def _():
            process(rows_a, rows_b, sem_a, sem_b, i)

        @pl.when(i % 2 == 1)
        def _():
            process(rows_b, rows_a, sem_b, sem_a, i)

        return _

    lax.fori_loop(0, CH, step, None)
    plsc.subcore_barrier()
    pltpu.sync_copy(acc_sp.at[pl.ds(s * ROWS_W, ROWS_W)],
                    accp_hbm.at[c, pl.ds(s * ROWS_W, ROWS_W)])

    @pl.when(s == NS - 1)
    def _():
        pltpu.sync_copy(acc_sp.at[pl.ds(NS * ROWS_W, ROWS_TAIL)],
                        accp_hbm.at[c, pl.ds(NS * ROWS_W, ROWS_TAIL)])


_edge_pass = pl.kernel(
    _edge_body,
    out_type=jax.ShapeDtypeStruct((NC, N, D), jnp.float32),
    mesh=_mesh(),
    scratch_types=[
        pltpu.VMEM_SHARED((N, D), jnp.float32),
        pltpu.VMEM((K, D), jnp.float32),
        pltpu.VMEM((K, D), jnp.float32),
        pltpu.VMEM((EW,), jnp.int32),
        pltpu.VMEM((K,), jnp.int32),
        pltpu.VMEM((EW,), jnp.float32),
        pltpu.SemaphoreType.DMA,
        pltpu.SemaphoreType.DMA,
    ],
    compiler_params=_SC_PARAMS,
)


BN = 1000


def _mm(a, b):
    return jnp.dot(a, b, preferred_element_type=jnp.float32)


def _transform(h, wroot_ref, b_ref, wr_ref, h_ref, out0_ref, hr_ref):
    h_ref[...] = h
    out0_ref[...] = _mm(h, wroot_ref[...]) + b_ref[...]
    for r in range(R):
        hr_ref[r] = _mm(h, wr_ref[r])


def _embed_body(x_ref, emb_ref, wroot_ref, b_ref, wr_ref, h_ref, out0_ref, hr_ref):
    xb = x_ref[...]
    oh = jnp.where(xb == lax.broadcasted_iota(jnp.int32, (BN, VOCAB), 1), 1.0, 0.0)
    h = _mm(oh, emb_ref[...])
    _transform(h, wroot_ref, b_ref, wr_ref, h_ref, out0_ref, hr_ref)


def _layer_body(hprev_ref, out0prev_ref, acc_ref, wroot_ref, b_ref, wr_ref,
                h_ref, out0_ref, hr_ref):
    conv = out0prev_ref[...] + acc_ref[0] + acc_ref[1]
    h = jnp.maximum(conv, 0.0) + hprev_ref[...]
    _transform(h, wroot_ref, b_ref, wr_ref, h_ref, out0_ref, hr_ref)


def _final_body(hprev_ref, out0_ref, acc_ref, o_ref):
    conv = out0_ref[...] + acc_ref[0] + acc_ref[1]
    o_ref[...] = jnp.maximum(conv, 0.0) + hprev_ref[...]


_full2 = pl.BlockSpec((VOCAB, D), lambda i: (0, 0))
_wroot_spec = pl.BlockSpec((D, D), lambda i: (0, 0))
_b_spec = pl.BlockSpec((1, D), lambda i: (0, 0))
_wr_spec = pl.BlockSpec((R, D, D), lambda i: (0, 0, 0))
_nd_spec = pl.BlockSpec((BN, D), lambda i: (i, 0))
_hr_spec = pl.BlockSpec((R, BN, D), lambda i: (0, i, 0))
_acc_spec = pl.BlockSpec((NC, BN, D), lambda i: (0, i, 0))
_x_spec = pl.BlockSpec((BN, 1), lambda i: (i, 0))

_nd_t = jax.ShapeDtypeStruct((N, D), jnp.float32)
_hr_t = jax.ShapeDtypeStruct((R, N, D), jnp.float32)


def _embed_transform(x32, emb, wroot, b2, wr):
    return pl.pallas_call(
        _embed_body,
        grid=(N // BN,),
        in_specs=[_x_spec, _full2, _wroot_spec, _b_spec, _wr_spec],
        out_specs=[_nd_spec, _nd_spec, _hr_spec],
        out_shape=[_nd_t, _nd_t, _hr_t],
    )(x32, emb, wroot, b2, wr)


def _layer_transform(hprev, out0prev, accp, wroot, b2, wr):
    return pl.pallas_call(
        _layer_body,
        grid=(N // BN,),
        in_specs=[_nd_spec, _nd_spec, _acc_spec, _wroot_spec, _b_spec, _wr_spec],
        out_specs=[_nd_spec, _nd_spec, _hr_spec],
        out_shape=[_nd_t, _nd_t, _hr_t],
    )(hprev, out0prev, accp, wroot, b2, wr)


def _final(hprev, out0, accp):
    return pl.pallas_call(
        _final_body,
        grid=(N // BN,),
        in_specs=[_nd_spec, _nd_spec, _acc_spec],
        out_specs=_nd_spec,
        out_shape=_nd_t,
    )(hprev, out0, accp)


def kernel(x, edge_index, edge_attr, emb, Wr1, Wroot1, b1, Wr2, Wroot2, b2,
           Wr3, Wroot3, b3):
    x32 = x.astype(jnp.int32)
    src = edge_index[0].astype(jnp.int32)
    dst = edge_index[1].astype(jnp.int32)
    rel = edge_attr.astype(jnp.int32)
    gidx = rel * N + src
    cidx = dst * R + rel
    didx3 = dst
    zeros_m = jnp.zeros((ROWS_W, D), jnp.float32)

    s_e = _scale_pass(cidx, zeros_m)

    h1, out0_1, hr1 = _embed_transform(x32, emb, Wroot1, b1.reshape(1, D), Wr1)
    acc1 = _edge_pass(hr1.reshape(R * N, D), gidx, didx3, s_e, zeros_m)
    h2, out0_2, hr2 = _layer_transform(h1, out0_1, acc1, Wroot2, b2.reshape(1, D), Wr2)
    acc2 = _edge_pass(hr2.reshape(R * N, D), gidx, didx3, s_e, zeros_m)
    h3, out0_3, hr3 = _layer_transform(h2, out0_2, acc2, Wroot3, b3.reshape(1, D), Wr3)
    acc3 = _edge_pass(hr3.reshape(R * N, D), gidx, didx3, s_e, zeros_m)
    return _final(h3, out0_3, acc3)


# speedup vs baseline: 22.9910x; 1.4963x over previous
import numpy as np

import jax
import jax.numpy as jnp
from jax import lax
from jax.experimental import pallas as pl
from jax.experimental.pallas import tpu as pltpu
from jax.experimental.pallas import tpu_sc as plsc

N = 10000
E = 320000
VOCAB = 64
D = 128
R = 4

NC = 2
NS = 16
L = 16
NW = NC * NS

K = 80
EW = E // NW
CH = EW // K
EA = E // NS
ROWS_W = 624
ROWS_TAIL = N - NS * ROWS_W
HROWS = 320
HR_W = 40

def _lane_iota():
    return lax.iota(jnp.int32, L)


def _mesh():
    return plsc.VectorSubcoreMesh(
        core_axis_name="c", subcore_axis_name="s", num_cores=NC, num_subcores=NS
    )


_SC_PARAMS = pltpu.CompilerParams(needs_layout_passes=False)


_TAKE_DN = lax.GatherDimensionNumbers(offset_dims=(), collapsed_slice_dims=(0,),
                                      start_index_map=(0,))


def _take16(vec16, j):
    idx = jnp.full((L, 1), j, jnp.int32)
    return lax.gather(vec16, idx, _TAKE_DN, slice_sizes=(1,),
                      mode=lax.GatherScatterMode.PROMISE_IN_BOUNDS)


def _scale_body(cidx_hbm, zeros_hbm, s_hbm, hist_sp, hist_v, stage_v, cidx_v,
                rowid_v, sv_v, sem):
    del sem
    c = lax.axis_index("c")
    s = lax.axis_index("s")

    @pl.when(c == 0)
    def _():
        pltpu.sync_copy(zeros_hbm.at[pl.ds(0, HROWS)], hist_v)

        @pl.when(s < NS // 2)
        def _():
            pltpu.sync_copy(zeros_hbm.at[pl.ds(0, HR_W)],
                            hist_sp.at[pl.ds(s * HR_W, HR_W)])

        ones = jnp.ones((L,), jnp.float32)
        pltpu.sync_copy(cidx_hbm.at[pl.ds(s * EA, EA)], cidx_v)

        def hist_step(i, _):
            for jj in range(K // L):
                c16 = cidx_v[pl.ds(i * K + jj * L, L)]
                plsc.addupdate_scatter(hist_v, [c16 >> 7, c16 & 127], ones)
            return _

        lax.fori_loop(0, EA // K, hist_step, None)

    plsc.subcore_barrier()

    @pl.when(c == 0)
    def _():
        for kk in range(HROWS // K):
            base = kk * K
            for jj in range(K // L):
                rowid_v[pl.ds(jj * L, L)] = _lane_iota() + (base + jj * L)
            pltpu.sync_copy(hist_v.at[pl.ds(base, K)], hist_sp.at[rowid_v],
                            add=True)

    plsc.subcore_barrier()

    @pl.when((c == 0) & (s < NS // 2))
    def _():
        off = s * HR_W
        pltpu.sync_copy(hist_sp.at[pl.ds(off, HR_W)], stage_v)
        for j in range(HR_W):
            for i8 in range(D // L):
                sl = pl.ds(i8 * L, L)
                stage_v[j, sl] = 1.0 / jnp.maximum(stage_v[j, sl], 1.0)
        pltpu.sync_copy(stage_v, hist_sp.at[pl.ds(off, HR_W)])

    plsc.subcore_barrier()

    @pl.when(c == 0)
    def _():
        pltpu.sync_copy(hist_sp, hist_v)

        def gather_step(i, _):
            for jj in range(K // L):
                c16 = cidx_v[pl.ds(i * K + jj * L, L)]
                sv_v[pl.ds(i * K + jj * L, L)] = plsc.load_gather(
                    hist_v, [c16 >> 7, c16 & 127])
            return _

        lax.fori_loop(0, EA // K, gather_step, None)
        pltpu.sync_copy(sv_v, s_hbm.at[pl.ds(s * EA, EA)])


_scale_pass = pl.kernel(
    _scale_body,
    out_type=jax.ShapeDtypeStruct((E,), jnp.float32),
    mesh=_mesh(),
    scratch_types=[
        pltpu.VMEM_SHARED((HROWS, D), jnp.float32),
        pltpu.VMEM((HROWS, D), jnp.float32),
        pltpu.VMEM((HR_W, D), jnp.float32),
        pltpu.VMEM((EA,), jnp.int32),
        pltpu.VMEM((K,), jnp.int32),
        pltpu.VMEM((EA,), jnp.float32),
        pltpu.SemaphoreType.DMA,
    ],
    compiler_params=_SC_PARAMS,
)


def _edge_body(hr_hbm, gidx_hbm, didx_hbm, s_hbm, zeros_hbm, accp_hbm,
               acc_sp, rows_a, rows_b, gidx_v, didx_a, didx_b, s_v,
               sem_a, sem_b, ssem_a, ssem_b):
    c = lax.axis_index("c")
    s = lax.axis_index("s")
    wid = c * NS + s

    pltpu.sync_copy(zeros_hbm.at[pl.ds(0, ROWS_W)], acc_sp.at[pl.ds(s * ROWS_W, ROWS_W)])

    @pl.when(s == NS - 1)
    def _():
        pltpu.sync_copy(zeros_hbm.at[pl.ds(0, ROWS_TAIL)],
                        acc_sp.at[pl.ds(NS * ROWS_W, ROWS_TAIL)])

    pltpu.sync_copy(gidx_hbm.at[pl.ds(wid * EW, EW)], gidx_v)
    pltpu.sync_copy(s_hbm.at[pl.ds(wid * EW, EW)], s_v)
    plsc.subcore_barrier()

    pltpu.async_copy(didx_hbm.at[pl.ds(wid * EW, K)], didx_a, sem_a)
    pltpu.async_copy(hr_hbm.at[gidx_v.at[pl.ds(0, K)]], rows_a, sem_a)

    def process(cur, nxt, dcur, dnxt, sem_cur, sem_nxt, ssem_cur, ssem_nxt, i):
        pltpu.make_async_copy(didx_hbm.at[pl.ds(0, K)], dcur, sem_cur).wait()
        pltpu.make_async_copy(zeros_hbm.at[pl.ds(0, K)], cur, sem_cur).wait()

        @pl.when(i >= 1)
        def _():
            pltpu.make_async_copy(zeros_hbm.at[pl.ds(0, K)], nxt,
                                  ssem_nxt).wait()

        @pl.when(i + 1 < CH)
        def _():
            base = wid * EW + (i + 1) * K
            pltpu.async_copy(didx_hbm.at[pl.ds(base, K)], dnxt, sem_nxt)
            pltpu.async_copy(hr_hbm.at[gidx_v.at[pl.ds((i + 1) * K, K)]],
                             nxt, sem_nxt)

        for jj in range(K // L):
            s16 = s_v[pl.ds(i * K + jj * L, L)]
            for t in range(L):
                sj = _take16(s16, t)
                for i8 in range(D // L):
                    sl = pl.ds(i8 * L, L)
                    cur[jj * L + t, sl] = cur[jj * L + t, sl] * sj
        pltpu.async_copy(cur, acc_sp.at[dcur], ssem_cur, add=True)

    def step(i, _):
        @pl.when(i % 2 == 0)
        def _():
            process(rows_a, rows_b, didx_a, didx_b, sem_a, sem_b,
                    ssem_a, ssem_b, i)

        @pl.when(i % 2 == 1)
        def _():
            process(rows_b, rows_a, didx_b, didx_a, sem_b, sem_a,
                    ssem_b, ssem_a, i)

        return _

    lax.fori_loop(0, CH, step, None)
    pltpu.make_async_copy(zeros_hbm.at[pl.ds(0, K)], rows_a, ssem_a).wait()
    plsc.subcore_barrier()
    pltpu.sync_copy(acc_sp.at[pl.ds(s * ROWS_W, ROWS_W)],
                    accp_hbm.at[c, pl.ds(s * ROWS_W, ROWS_W)])

    @pl.when(s == NS - 1)
    def _():
        pltpu.sync_copy(acc_sp.at[pl.ds(NS * ROWS_W, ROWS_TAIL)],
                        accp_hbm.at[c, pl.ds(NS * ROWS_W, ROWS_TAIL)])


_edge_pass = pl.kernel(
    _edge_body,
    out_type=jax.ShapeDtypeStruct((NC, N, D), jnp.float32),
    mesh=_mesh(),
    scratch_types=[
        pltpu.VMEM_SHARED((N, D), jnp.float32),
        pltpu.VMEM((K, D), jnp.float32),
        pltpu.VMEM((K, D), jnp.float32),
        pltpu.VMEM((EW,), jnp.int32),
        pltpu.VMEM((K,), jnp.int32),
        pltpu.VMEM((K,), jnp.int32),
        pltpu.VMEM((EW,), jnp.float32),
        pltpu.SemaphoreType.DMA,
        pltpu.SemaphoreType.DMA,
        pltpu.SemaphoreType.DMA,
        pltpu.SemaphoreType.DMA,
    ],
    compiler_params=_SC_PARAMS,
)


BN = 1000


def _mm(a, b):
    return jnp.dot(a, b, preferred_element_type=jnp.float32)


def _transform(h, wroot_ref, b_ref, wr_ref, h_ref, out0_ref, hr_ref):
    h_ref[...] = h
    out0_ref[...] = _mm(h, wroot_ref[...]) + b_ref[...]
    for r in range(R):
        hr_ref[r] = _mm(h, wr_ref[r])


def _embed_body(x_ref, emb_ref, wroot_ref, b_ref, wr_ref, h_ref, out0_ref, hr_ref):
    xb = x_ref[...]
    oh = jnp.where(xb == lax.broadcasted_iota(jnp.int32, (BN, VOCAB), 1), 1.0, 0.0)
    h = _mm(oh, emb_ref[...])
    _transform(h, wroot_ref, b_ref, wr_ref, h_ref, out0_ref, hr_ref)


def _layer_body(hprev_ref, out0prev_ref, acc_ref, wroot_ref, b_ref, wr_ref,
                h_ref, out0_ref, hr_ref):
    conv = out0prev_ref[...] + acc_ref[0] + acc_ref[1]
    h = jnp.maximum(conv, 0.0) + hprev_ref[...]
    _transform(h, wroot_ref, b_ref, wr_ref, h_ref, out0_ref, hr_ref)


def _final_body(hprev_ref, out0_ref, acc_ref, o_ref):
    conv = out0_ref[...] + acc_ref[0] + acc_ref[1]
    o_ref[...] = jnp.maximum(conv, 0.0) + hprev_ref[...]


_full2 = pl.BlockSpec((VOCAB, D), lambda i: (0, 0))
_wroot_spec = pl.BlockSpec((D, D), lambda i: (0, 0))
_b_spec = pl.BlockSpec((1, D), lambda i: (0, 0))
_wr_spec = pl.BlockSpec((R, D, D), lambda i: (0, 0, 0))
_nd_spec = pl.BlockSpec((BN, D), lambda i: (i, 0))
_hr_spec = pl.BlockSpec((R, BN, D), lambda i: (0, i, 0))
_acc_spec = pl.BlockSpec((NC, BN, D), lambda i: (0, i, 0))
_x_spec = pl.BlockSpec((BN, 1), lambda i: (i, 0))

_nd_t = jax.ShapeDtypeStruct((N, D), jnp.float32)
_hr_t = jax.ShapeDtypeStruct((R, N, D), jnp.float32)


def _embed_transform(x32, emb, wroot, b2, wr):
    return pl.pallas_call(
        _embed_body,
        grid=(N // BN,),
        in_specs=[_x_spec, _full2, _wroot_spec, _b_spec, _wr_spec],
        out_specs=[_nd_spec, _nd_spec, _hr_spec],
        out_shape=[_nd_t, _nd_t, _hr_t],
    )(x32, emb, wroot, b2, wr)


def _layer_transform(hprev, out0prev, accp, wroot, b2, wr):
    return pl.pallas_call(
        _layer_body,
        grid=(N // BN,),
        in_specs=[_nd_spec, _nd_spec, _acc_spec, _wroot_spec, _b_spec, _wr_spec],
        out_specs=[_nd_spec, _nd_spec, _hr_spec],
        out_shape=[_nd_t, _nd_t, _hr_t],
    )(hprev, out0prev, accp, wroot, b2, wr)


def _final(hprev, out0, accp):
    return pl.pallas_call(
        _final_body,
        grid=(N // BN,),
        in_specs=[_nd_spec, _nd_spec, _acc_spec],
        out_specs=_nd_spec,
        out_shape=_nd_t,
    )(hprev, out0, accp)


def kernel(x, edge_index, edge_attr, emb, Wr1, Wroot1, b1, Wr2, Wroot2, b2,
           Wr3, Wroot3, b3):
    x32 = x.astype(jnp.int32)
    src = edge_index[0].astype(jnp.int32)
    dst = edge_index[1].astype(jnp.int32)
    rel = edge_attr.astype(jnp.int32)
    gidx = rel * N + src
    cidx = dst * R + rel
    didx3 = dst
    zeros_m = jnp.zeros((ROWS_W, D), jnp.float32)

    s_e = _scale_pass(cidx, zeros_m)

    h1, out0_1, hr1 = _embed_transform(x32, emb, Wroot1, b1.reshape(1, D), Wr1)
    acc1 = _edge_pass(hr1.reshape(R * N, D), gidx, didx3, s_e, zeros_m)
    h2, out0_2, hr2 = _layer_transform(h1, out0_1, acc1, Wroot2, b2.reshape(1, D), Wr2)
    acc2 = _edge_pass(hr2.reshape(R * N, D), gidx, didx3, s_e, zeros_m)
    h3, out0_3, hr3 = _layer_transform(h2, out0_2, acc2, Wroot3, b3.reshape(1, D), Wr3)
    acc3 = _edge_pass(hr3.reshape(R * N, D), gidx, didx3, s_e, zeros_m)
    return _final(h3, out0_3, acc3)


# speedup vs baseline: 23.0056x; 1.0006x over previous
import jax
import jax.numpy as jnp
from jax import lax
from jax.experimental import pallas as pl
from jax.experimental.pallas import tpu as pltpu
from jax.experimental.pallas import tpu_sc as plsc

N = 10000
E = 320000
VOCAB = 64
D = 128
R = 4

NC = 2
NS = 16
L = 16
NW = NC * NS

K = 80
EW = E // NW
CH = EW // K
EA = E // NS
ROWS_W = 624
ROWS_TAIL = N - NS * ROWS_W
HROWS = 320
HR_W = 40

def _lane_iota():
    return lax.iota(jnp.int32, L)


def _mesh():
    return plsc.VectorSubcoreMesh(
        core_axis_name="c", subcore_axis_name="s", num_cores=NC, num_subcores=NS
    )


_SC_PARAMS = pltpu.CompilerParams(needs_layout_passes=False)


_TAKE_DN = lax.GatherDimensionNumbers(offset_dims=(), collapsed_slice_dims=(0,),
                                      start_index_map=(0,))


def _take16(vec16, j):
    idx = jnp.full((L, 1), j, jnp.int32)
    return lax.gather(vec16, idx, _TAKE_DN, slice_sizes=(1,),
                      mode=lax.GatherScatterMode.PROMISE_IN_BOUNDS)


def _scale_body(cidx_hbm, zeros_hbm, s_hbm, hist_sp, hist_v, stage_v, cidx_v,
                rowid_v, sv_v, sem):
    del sem
    c = lax.axis_index("c")
    s = lax.axis_index("s")

    @pl.when(c == 0)
    def _():
        pltpu.sync_copy(zeros_hbm.at[pl.ds(0, HROWS)], hist_v)

        @pl.when(s < NS // 2)
        def _():
            pltpu.sync_copy(zeros_hbm.at[pl.ds(0, HR_W)],
                            hist_sp.at[pl.ds(s * HR_W, HR_W)])

        ones = jnp.ones((L,), jnp.float32)
        pltpu.sync_copy(cidx_hbm.at[pl.ds(s * EA, EA)], cidx_v)

        def hist_step(i, _):
            for jj in range(K // L):
                c16 = cidx_v[pl.ds(i * K + jj * L, L)]
                plsc.addupdate_scatter(hist_v, [c16 >> 7, c16 & 127], ones)
            return _

        lax.fori_loop(0, EA // K, hist_step, None)

    plsc.subcore_barrier()

    @pl.when(c == 0)
    def _():
        for kk in range(HROWS // K):
            base = kk * K
            for jj in range(K // L):
                rowid_v[pl.ds(jj * L, L)] = _lane_iota() + (base + jj * L)
            pltpu.sync_copy(hist_v.at[pl.ds(base, K)], hist_sp.at[rowid_v],
                            add=True)

    plsc.subcore_barrier()

    @pl.when((c == 0) & (s < NS // 2))
    def _():
        off = s * HR_W
        pltpu.sync_copy(hist_sp.at[pl.ds(off, HR_W)], stage_v)
        for j in range(HR_W):
            for i8 in range(D // L):
                sl = pl.ds(i8 * L, L)
                stage_v[j, sl] = 1.0 / jnp.maximum(stage_v[j, sl], 1.0)
        pltpu.sync_copy(stage_v, hist_sp.at[pl.ds(off, HR_W)])

    plsc.subcore_barrier()

    @pl.when(c == 0)
    def _():
        pltpu.sync_copy(hist_sp, hist_v)

        def gather_step(i, _):
            for jj in range(K // L):
                c16 = cidx_v[pl.ds(i * K + jj * L, L)]
                sv_v[pl.ds(i * K + jj * L, L)] = plsc.load_gather(
                    hist_v, [c16 >> 7, c16 & 127])
            return _

        lax.fori_loop(0, EA // K, gather_step, None)
        pltpu.sync_copy(sv_v, s_hbm.at[pl.ds(s * EA, EA)])


_scale_pass = pl.kernel(
    _scale_body,
    out_type=jax.ShapeDtypeStruct((E,), jnp.float32),
    mesh=_mesh(),
    scratch_types=[
        pltpu.VMEM_SHARED((HROWS, D), jnp.float32),
        pltpu.VMEM((HROWS, D), jnp.float32),
        pltpu.VMEM((HR_W, D), jnp.float32),
        pltpu.VMEM((EA,), jnp.int32),
        pltpu.VMEM((K,), jnp.int32),
        pltpu.VMEM((EA,), jnp.float32),
        pltpu.SemaphoreType.DMA,
    ],
    compiler_params=_SC_PARAMS,
)


def _edge_body(hr_hbm, gidx_hbm, didx_hbm, s_hbm, zeros_hbm, accp_hbm,
               acc_sp, rows_a, rows_b, gidx_v, didx_a, didx_b, s_v,
               sem_a, sem_b, ssem_a, ssem_b):
    c = lax.axis_index("c")
    s = lax.axis_index("s")
    wid = c * NS + s

    pltpu.sync_copy(zeros_hbm.at[pl.ds(0, ROWS_W)], acc_sp.at[pl.ds(s * ROWS_W, ROWS_W)])

    @pl.when(s == NS - 1)
    def _():
        pltpu.sync_copy(zeros_hbm.at[pl.ds(0, ROWS_TAIL)],
                        acc_sp.at[pl.ds(NS * ROWS_W, ROWS_TAIL)])

    pltpu.sync_copy(gidx_hbm.at[pl.ds(wid * EW, EW)], gidx_v)
    pltpu.sync_copy(s_hbm.at[pl.ds(wid * EW, EW)], s_v)
    plsc.subcore_barrier()

    pltpu.async_copy(didx_hbm.at[pl.ds(wid * EW, K)], didx_a, sem_a)
    pltpu.async_copy(hr_hbm.at[gidx_v.at[pl.ds(0, K)]], rows_a, sem_a)

    def process(cur, nxt, dcur, dnxt, sem_cur, sem_nxt, ssem_cur, ssem_nxt, i):
        pltpu.make_async_copy(didx_hbm.at[pl.ds(0, K)], dcur, sem_cur).wait()
        pltpu.make_async_copy(zeros_hbm.at[pl.ds(0, K)], cur, sem_cur).wait()

        @pl.when(i >= 1)
        def _():
            pltpu.make_async_copy(zeros_hbm.at[pl.ds(0, K)], nxt,
                                  ssem_nxt).wait()

        @pl.when(i + 1 < CH)
        def _():
            base = wid * EW + (i + 1) * K
            pltpu.async_copy(didx_hbm.at[pl.ds(base, K)], dnxt, sem_nxt)
            pltpu.async_copy(hr_hbm.at[gidx_v.at[pl.ds((i + 1) * K, K)]],
                             nxt, sem_nxt)

        for jj in range(K // L):
            s16 = s_v[pl.ds(i * K + jj * L, L)]
            for t in range(L):
                sj = _take16(s16, t)
                for i8 in range(D // L):
                    sl = pl.ds(i8 * L, L)
                    cur[jj * L + t, sl] = cur[jj * L + t, sl] * sj
        pltpu.async_copy(cur, acc_sp.at[dcur], ssem_cur, add=True)

    def step(i, _):
        @pl.when(i % 2 == 0)
        def _():
            process(rows_a, rows_b, didx_a, didx_b, sem_a, sem_b,
                    ssem_a, ssem_b, i)

        @pl.when(i % 2 == 1)
        def _():
            process(rows_b, rows_a, didx_b, didx_a, sem_b, sem_a,
                    ssem_b, ssem_a, i)

        return _

    lax.fori_loop(0, CH, step, None)
    pltpu.make_async_copy(zeros_hbm.at[pl.ds(0, K)], rows_a, ssem_a).wait()
    plsc.subcore_barrier()
    pltpu.sync_copy(acc_sp.at[pl.ds(s * ROWS_W, ROWS_W)],
                    accp_hbm.at[c, pl.ds(s * ROWS_W, ROWS_W)])

    @pl.when(s == NS - 1)
    def _():
        pltpu.sync_copy(acc_sp.at[pl.ds(NS * ROWS_W, ROWS_TAIL)],
                        accp_hbm.at[c, pl.ds(NS * ROWS_W, ROWS_TAIL)])


_edge_pass = pl.kernel(
    _edge_body,
    out_type=jax.ShapeDtypeStruct((NC, N, D), jnp.float32),
    mesh=_mesh(),
    scratch_types=[
        pltpu.VMEM_SHARED((N, D), jnp.float32),
        pltpu.VMEM((K, D), jnp.float32),
        pltpu.VMEM((K, D), jnp.float32),
        pltpu.VMEM((EW,), jnp.int32),
        pltpu.VMEM((K,), jnp.int32),
        pltpu.VMEM((K,), jnp.int32),
        pltpu.VMEM((EW,), jnp.float32),
        pltpu.SemaphoreType.DMA,
        pltpu.SemaphoreType.DMA,
        pltpu.SemaphoreType.DMA,
        pltpu.SemaphoreType.DMA,
    ],
    compiler_params=_SC_PARAMS,
)


BN = 1000


def _mm(a, b):
    return jnp.dot(a, b, preferred_element_type=jnp.float32)


def _transform(h, wroot_ref, b_ref, wr_ref, h_ref, out0_ref, hr_ref):
    h_ref[...] = h
    out0_ref[...] = _mm(h, wroot_ref[...]) + b_ref[...]
    for r in range(R):
        hr_ref[r] = _mm(h, wr_ref[r])


def _embed_body(x_ref, emb_ref, wroot_ref, b_ref, wr_ref, h_ref, out0_ref, hr_ref):
    xb = x_ref[...]
    oh = jnp.where(xb == lax.broadcasted_iota(jnp.int32, (BN, VOCAB), 1), 1.0, 0.0)
    h = _mm(oh, emb_ref[...])
    _transform(h, wroot_ref, b_ref, wr_ref, h_ref, out0_ref, hr_ref)


def _layer_body(hprev_ref, out0prev_ref, acc_ref, wroot_ref, b_ref, wr_ref,
                h_ref, out0_ref, hr_ref):
    conv = out0prev_ref[...] + acc_ref[0] + acc_ref[1]
    h = jnp.maximum(conv, 0.0) + hprev_ref[...]
    _transform(h, wroot_ref, b_ref, wr_ref, h_ref, out0_ref, hr_ref)


def _final_body(hprev_ref, out0_ref, acc_ref, o_ref):
    conv = out0_ref[...] + acc_ref[0] + acc_ref[1]
    o_ref[...] = jnp.maximum(conv, 0.0) + hprev_ref[...]


_full2 = pl.BlockSpec((VOCAB, D), lambda i: (0, 0))
_wroot_spec = pl.BlockSpec((D, D), lambda i: (0, 0))
_b_spec = pl.BlockSpec((1, D), lambda i: (0, 0))
_wr_spec = pl.BlockSpec((R, D, D), lambda i: (0, 0, 0))
_nd_spec = pl.BlockSpec((BN, D), lambda i: (i, 0))
_hr_spec = pl.BlockSpec((R, BN, D), lambda i: (0, i, 0))
_acc_spec = pl.BlockSpec((NC, BN, D), lambda i: (0, i, 0))
_x_spec = pl.BlockSpec((BN, 1), lambda i: (i, 0))

_nd_t = jax.ShapeDtypeStruct((N, D), jnp.float32)
_hr_t = jax.ShapeDtypeStruct((R, N, D), jnp.float32)


def _embed_transform(x32, emb, wroot, b2, wr):
    return pl.pallas_call(
        _embed_body,
        grid=(N // BN,),
        in_specs=[_x_spec, _full2, _wroot_spec, _b_spec, _wr_spec],
        out_specs=[_nd_spec, _nd_spec, _hr_spec],
        out_shape=[_nd_t, _nd_t, _hr_t],
    )(x32, emb, wroot, b2, wr)


def _layer_transform(hprev, out0prev, accp, wroot, b2, wr):
    return pl.pallas_call(
        _layer_body,
        grid=(N // BN,),
        in_specs=[_nd_spec, _nd_spec, _acc_spec, _wroot_spec, _b_spec, _wr_spec],
        out_specs=[_nd_spec, _nd_spec, _hr_spec],
        out_shape=[_nd_t, _nd_t, _hr_t],
    )(hprev, out0prev, accp, wroot, b2, wr)


def _final(hprev, out0, accp):
    return pl.pallas_call(
        _final_body,
        grid=(N // BN,),
        in_specs=[_nd_spec, _nd_spec, _acc_spec],
        out_specs=_nd_spec,
        out_shape=_nd_t,
    )(hprev, out0, accp)


def kernel(x, edge_index, edge_attr, emb, Wr1, Wroot1, b1, Wr2, Wroot2, b2,
           Wr3, Wroot3, b3):
    x32 = x.astype(jnp.int32)
    src = edge_index[0].astype(jnp.int32)
    dst = edge_index[1].astype(jnp.int32)
    rel = edge_attr.astype(jnp.int32)
    gidx = rel * N + src
    cidx = dst * R + rel
    zeros_m = jnp.zeros((ROWS_W, D), jnp.float32)

    s_e = _scale_pass(cidx, zeros_m)

    h1, out0_1, hr1 = _embed_transform(x32, emb, Wroot1, b1.reshape(1, D), Wr1)
    acc1 = _edge_pass(hr1.reshape(R * N, D), gidx, dst, s_e, zeros_m)
    h2, out0_2, hr2 = _layer_transform(h1, out0_1, acc1, Wroot2, b2.reshape(1, D), Wr2)
    acc2 = _edge_pass(hr2.reshape(R * N, D), gidx, dst, s_e, zeros_m)
    h3, out0_3, hr3 = _layer_transform(h2, out0_2, acc2, Wroot3, b3.reshape(1, D), Wr3)
    acc3 = _edge_pass(hr3.reshape(R * N, D), gidx, dst, s_e, zeros_m)
    return _final(h3, out0_3, acc3)
